# baseline (speedup 1.0000x reference)
import functools

import jax
import jax.numpy as jnp
from jax import lax
from jax.experimental import pallas as pl
from jax.experimental.pallas import tpu as pltpu
from jax.experimental.pallas import tpu_sc as plsc

N = 10000
E = 320000
D = 128
G = 64

NC = 2
NS = 16
NW = NC * NS
L = 16

EW = E // NW
KC = 2000
RB = 80
ZR = 125

_mesh = plsc.VectorSubcoreMesh(core_axis_name="c", subcore_axis_name="s")


@functools.partial(
    pl.kernel,
    out_type=[
        jax.ShapeDtypeStruct((NW * N,), jnp.float32),
        jax.ShapeDtypeStruct((NW * 3 * N,), jnp.float32),
    ],
    mesh=_mesh,
    compiler_params=pltpu.CompilerParams(needs_layout_passes=False),
    scratch_types=[
        pltpu.VMEM((N,), jnp.float32),
        pltpu.VMEM((N,), jnp.float32),
        pltpu.VMEM((N,), jnp.float32),
        pltpu.VMEM((N,), jnp.float32),
        pltpu.VMEM((KC,), jnp.int32),
        pltpu.VMEM((KC,), jnp.float32),
        pltpu.VMEM((KC,), jnp.float32),
        pltpu.VMEM((KC,), jnp.float32),
    ],
)
def _sc_prologue(dst_hbm, ae1_hbm, ae2_hbm, ae3_hbm, degp_out, aesp_out,
                 deg_v, s1_v, s2_v, s3_v, dstc, a1c, a2c, a3c):
    c = lax.axis_index("c")
    s = lax.axis_index("s")
    w = s * NC + c
    base = w * EW
    zf = jnp.zeros((L,), jnp.float32)

    def zb(i, carry):
        deg_v[pl.ds(i * L, L)] = zf
        s1_v[pl.ds(i * L, L)] = zf
        s2_v[pl.ds(i * L, L)] = zf
        s3_v[pl.ds(i * L, L)] = zf
        return carry

    lax.fori_loop(0, N // L, zb, 0)

    ones = jnp.ones((L,), jnp.float32)

    def chunk(k, carry):
        cb = base + k * KC
        pltpu.sync_copy(dst_hbm.at[pl.ds(cb, KC)], dstc)
        pltpu.sync_copy(ae1_hbm.at[pl.ds(cb, KC)], a1c)
        pltpu.sync_copy(ae2_hbm.at[pl.ds(cb, KC)], a2c)
        pltpu.sync_copy(ae3_hbm.at[pl.ds(cb, KC)], a3c)

        def body(j, c2):
            sl = pl.ds(j * L, L)
            idx = dstc[sl]
            plsc.addupdate_scatter(deg_v, [idx], ones)
            plsc.addupdate_scatter(s1_v, [idx], a1c[sl])
            plsc.addupdate_scatter(s2_v, [idx], a2c[sl])
            plsc.addupdate_scatter(s3_v, [idx], a3c[sl])
            return c2

        lax.fori_loop(0, KC // L, body, 0, unroll=2)
        return carry

    lax.fori_loop(0, EW // KC, chunk, 0)
    pltpu.sync_copy(deg_v, degp_out.at[pl.ds(w * N, N)])
    pltpu.sync_copy(s1_v, aesp_out.at[pl.ds((0 * NW + w) * N, N)])
    pltpu.sync_copy(s2_v, aesp_out.at[pl.ds((1 * NW + w) * N, N)])
    pltpu.sync_copy(s3_v, aesp_out.at[pl.ds((2 * NW + w) * N, N)])


@functools.partial(
    pl.kernel,
    out_type=[
        jax.ShapeDtypeStruct((NW * N,), jnp.float32),
        jax.ShapeDtypeStruct((E,), jnp.float32),
    ],
    mesh=_mesh,
    compiler_params=pltpu.CompilerParams(needs_layout_passes=False),
    scratch_types=[
        pltpu.VMEM((N,), jnp.float32),
        pltpu.VMEM((N,), jnp.float32),
        pltpu.VMEM((N,), jnp.float32),
        pltpu.VMEM((KC,), jnp.int32),
        pltpu.VMEM((KC,), jnp.int32),
        pltpu.VMEM((KC,), jnp.float32),
        pltpu.VMEM((KC,), jnp.float32),
    ],
)
def _sc_scalar(src_hbm, dst_hbm, ae_hbm, asv_hbm, adv_hbm,
               denp_out, exq_out,
               asv_v, adv_v, den_v, srcc, dstc, aec, exc):
    c = lax.axis_index("c")
    s = lax.axis_index("s")
    w = s * NC + c
    base = w * EW
    zf = jnp.zeros((L,), jnp.float32)

    pltpu.sync_copy(asv_hbm, asv_v)
    pltpu.sync_copy(adv_hbm, adv_v)

    def zb(i, carry):
        den_v[pl.ds(i * L, L)] = zf
        return carry

    lax.fori_loop(0, N // L, zb, 0)

    def chunk(k, carry):
        cb = base + k * KC
        pltpu.sync_copy(src_hbm.at[pl.ds(cb, KC)], srcc)
        pltpu.sync_copy(dst_hbm.at[pl.ds(cb, KC)], dstc)
        pltpu.sync_copy(ae_hbm.at[pl.ds(cb, KC)], aec)

        def sbody(j, c2):
            sl = pl.ds(j * L, L)
            di = dstc[sl]
            a = plsc.load_gather(asv_v, [srcc[sl]])
            b = plsc.load_gather(adv_v, [di])
            al = a + b + aec[sl]
            al = jnp.where(al >= 0, al, 0.2 * al)
            ex = jnp.exp(al)
            exc[sl] = ex
            plsc.addupdate_scatter(den_v, [di], ex)
            return c2

        lax.fori_loop(0, KC // L, sbody, 0, unroll=2)
        pltpu.sync_copy(exc, exq_out.at[pl.ds(cb, KC)])
        return carry

    lax.fori_loop(0, EW // KC, chunk, 0)
    pltpu.sync_copy(den_v, denp_out.at[pl.ds(w * N, N)])


NBAT = KC // RB


@functools.partial(
    pl.kernel,
    out_type=[
        jax.ShapeDtypeStruct((NC, N, D), jnp.float32),
    ],
    mesh=_mesh,
    compiler_params=pltpu.CompilerParams(needs_layout_passes=False),
    scratch_types=[
        pltpu.VMEM_SHARED((N, D), jnp.float32),
        pltpu.VMEM((KC,), jnp.int32),
        pltpu.VMEM((KC,), jnp.int32),
        pltpu.VMEM((KC,), jnp.float32),
        pltpu.VMEM((RB,), jnp.int32),
        pltpu.VMEM((RB,), jnp.int32),
        pltpu.VMEM((RB, D), jnp.float32),
        pltpu.VMEM((RB, D), jnp.float32),
        pltpu.SemaphoreType.DMA,
        pltpu.SemaphoreType.DMA,
    ],
)
def _sc_rows(src_hbm, dst_hbm, exq_hbm, xh_hbm,
             accp_out,
             acc_sh, srcc, dstc, exc, idxd0, idxd1, rows0, rows1,
             sem0, sem1):
    c = lax.axis_index("c")
    s = lax.axis_index("s")
    w = s * NC + c
    base = w * EW
    zf = jnp.zeros((L,), jnp.float32)
    bufs = ((idxd0, rows0, sem0), (idxd1, rows1, sem1))

    def zr(i, carry):
        for q in range(D // L):
            rows0[i, pl.ds(q * L, L)] = zf
        return carry

    lax.fori_loop(0, RB, zr, 0)

    def zs(i, carry):
        pltpu.sync_copy(rows0, acc_sh.at[pl.ds(s * 624 + i * RB, RB)])
        return carry

    lax.fori_loop(0, 7, zs, 0)
    pltpu.sync_copy(rows0.at[pl.ds(0, 64)],
                    acc_sh.at[pl.ds(s * 624 + 560, 64)])

    @pl.when(s == NS - 1)
    def _ztail():
        pltpu.sync_copy(rows0.at[pl.ds(0, 16)], acc_sh.at[pl.ds(9984, 16)])

    plsc.subcore_barrier()

    def _issue(r, b):
        _, rows, sem = bufs[b]
        pltpu.async_copy(xh_hbm.at[srcc.at[pl.ds(r * RB, RB)]], rows, sem)

    def _finish(r, b):
        idxd, rows, sem = bufs[b]
        pltpu.make_async_copy(
            xh_hbm.at[srcc.at[pl.ds(r * RB, RB)]], rows, sem).wait()

        def cp(i, carry):
            sl = pl.ds(i * L, L)
            idxd[sl] = dstc[pl.ds(r * RB + i * L, L)]
            return carry

        lax.fori_loop(0, RB // L, cp, 0)

        def scale(rr, carry):
            exv = exc[pl.ds(r * RB + rr * L, L)]
            for j in range(L):
                wgt = exv[j]
                for q in range(D // L):
                    sl = pl.ds(q * L, L)
                    rows[rr * L + j, sl] = rows[rr * L + j, sl] * wgt
            return carry

        lax.fori_loop(0, RB // L, scale, 0)
        pltpu.sync_copy(rows, acc_sh.at[idxd], add=True)

    def chunk(k, carry):
        cb = base + k * KC
        pltpu.sync_copy(src_hbm.at[pl.ds(cb, KC)], srcc)
        pltpu.sync_copy(dst_hbm.at[pl.ds(cb, KC)], dstc)
        pltpu.sync_copy(exq_hbm.at[pl.ds(cb, KC)], exc)
        _issue(0, 0)

        def pair(r2, c2):
            r0 = 2 * r2

            @pl.when(r0 + 1 < NBAT)
            def _i1():
                _issue(r0 + 1, 1)

            _finish(r0, 0)

            @pl.when(r0 + 2 < NBAT)
            def _i0():
                _issue(r0 + 2, 0)

            @pl.when(r0 + 1 < NBAT)
            def _f1():
                _finish(r0 + 1, 1)

            return c2

        lax.fori_loop(0, (NBAT + 1) // 2, pair, 0)
        return carry

    lax.fori_loop(0, EW // KC, chunk, 0)
    plsc.subcore_barrier()

    def wb(i, carry):
        off = s * 624 + i * RB
        pltpu.sync_copy(acc_sh.at[pl.ds(off, RB)],
                        accp_out.at[c, pl.ds(off, RB)])
        return carry

    lax.fori_loop(0, 7, wb, 0)
    pltpu.sync_copy(acc_sh.at[pl.ds(s * 624 + 560, 64)],
                    accp_out.at[c, pl.ds(s * 624 + 560, 64)])

    @pl.when(s == NS - 1)
    def _wtail():
        pltpu.sync_copy(acc_sh.at[pl.ds(9984, 16)],
                        accp_out.at[c, pl.ds(9984, 16)])


BE = 8000
BN = 2000


def _edge_logits_body(ea_ref, We_ref, be_ref, Vp_ref, out_ref):
    ea = ea_ref[...] @ We_ref[...] + be_ref[...]
    ea = jnp.where(ea >= 0, ea, 0.01 * ea)
    out_ref[...] = ea @ Vp_ref[...]


def _edge_logits(edge_attr, We, be2, Vp):
    return pl.pallas_call(
        _edge_logits_body,
        grid=(E // BE,),
        in_specs=[
            pl.BlockSpec((BE, 16), lambda i: (i, 0)),
            pl.BlockSpec((16, D), lambda i: (0, 0)),
            pl.BlockSpec((1, D), lambda i: (0, 0)),
            pl.BlockSpec((D, 8), lambda i: (0, 0)),
        ],
        out_specs=pl.BlockSpec((BE, 8), lambda i: (i, 0)),
        out_shape=jax.ShapeDtypeStruct((E, 8), jnp.float32),
    )(edge_attr, We, be2, Vp)


def _proj0_body(x_ref, W_ref, b_ref, out_ref):
    h = x_ref[...] @ W_ref[...] + b_ref[...]
    out_ref[...] = jnp.where(h >= 0, h, 0.01 * h)


def _proj0(x, Wn, bn2):
    return pl.pallas_call(
        _proj0_body,
        grid=(N // BN,),
        in_specs=[
            pl.BlockSpec((BN, D), lambda i: (i, 0)),
            pl.BlockSpec((D, D), lambda i: (0, 0)),
            pl.BlockSpec((1, D), lambda i: (0, 0)),
        ],
        out_specs=pl.BlockSpec((BN, D), lambda i: (i, 0)),
        out_shape=jax.ShapeDtypeStruct((N, D), jnp.float32),
    )(x, Wn, bn2)


def _projL_body(h_ref, W_ref, A8_ref, xh_ref, s8_ref):
    xh = h_ref[...] @ W_ref[...]
    xh_ref[...] = xh
    s8_ref[...] = xh @ A8_ref[...]


def _projL(h, W, A8):
    return pl.pallas_call(
        _projL_body,
        grid=(N // BN,),
        in_specs=[
            pl.BlockSpec((BN, D), lambda i: (i, 0)),
            pl.BlockSpec((D, D), lambda i: (0, 0)),
            pl.BlockSpec((D, 8), lambda i: (0, 0)),
        ],
        out_specs=[
            pl.BlockSpec((BN, D), lambda i: (i, 0)),
            pl.BlockSpec((BN, 8), lambda i: (i, 0)),
        ],
        out_shape=[
            jax.ShapeDtypeStruct((N, D), jnp.float32),
            jax.ShapeDtypeStruct((N, 8), jnp.float32),
        ],
    )(h, W, A8)


def _epilogue_body(accp_ref, denpT_ref, degpT_ref, alpT_ref, xh_ref, s8_ref,
                   b_ref, out_ref):
    deg = jnp.maximum(jnp.sum(degpT_ref[...], axis=1, keepdims=True), 1.0)
    la = jnp.sum(alpT_ref[...], axis=1, keepdims=True) / deg
    zl = s8_ref[:, 0:1] + s8_ref[:, 1:2] + la
    exl = jnp.exp(jnp.where(zl >= 0, zl, 0.2 * zl))
    den = jnp.sum(denpT_ref[...], axis=1, keepdims=True) + exl
    acc = accp_ref[0] + accp_ref[1] + exl * xh_ref[...]
    h = acc / (den + 1e-16) + b_ref[...]
    out_ref[...] = jnp.where(h >= 0, h, 0.01 * h)


def _epilogue(accp, denpT, degpT, alpT, xh, s8, b2):
    return pl.pallas_call(
        _epilogue_body,
        grid=(N // BN,),
        in_specs=[
            pl.BlockSpec((NC, BN, D), lambda i: (0, i, 0)),
            pl.BlockSpec((BN, NW), lambda i: (i, 0)),
            pl.BlockSpec((BN, NW), lambda i: (i, 0)),
            pl.BlockSpec((BN, NW), lambda i: (i, 0)),
            pl.BlockSpec((BN, D), lambda i: (i, 0)),
            pl.BlockSpec((BN, 8), lambda i: (i, 0)),
            pl.BlockSpec((1, D), lambda i: (0, 0)),
        ],
        out_specs=pl.BlockSpec((BN, D), lambda i: (i, 0)),
        out_shape=jax.ShapeDtypeStruct((N, D), jnp.float32),
    )(accp, denpT, degpT, alpT, xh, s8, b2)


def _pool_body(h_ref, bf_ref, linW_ref, linb_ref, out_ref, acc_ref):
    i = pl.program_id(0)

    @pl.when(i == 0)
    def _init():
        acc_ref[...] = jnp.zeros_like(acc_ref)

    gids = jax.lax.broadcasted_iota(jnp.int32, (1, G), 1).astype(jnp.float32)
    mask = jnp.where(bf_ref[:, 0:1] == gids, 1.0, 0.0)
    acc_ref[...] += jax.lax.dot_general(
        mask, h_ref[...], (((0,), (0,)), ((), ())),
        preferred_element_type=jnp.float32)

    @pl.when(i == N // BN - 1)
    def _fin():
        out_ref[...] = acc_ref[...] @ linW_ref[...] + linb_ref[...]


def _pool(h, batchf, lin_W, linb2):
    return pl.pallas_call(
        _pool_body,
        grid=(N // BN,),
        in_specs=[
            pl.BlockSpec((BN, D), lambda i: (i, 0)),
            pl.BlockSpec((BN, 8), lambda i: (i, 0)),
            pl.BlockSpec((D, D), lambda i: (0, 0)),
            pl.BlockSpec((1, D), lambda i: (0, 0)),
        ],
        out_specs=pl.BlockSpec((G, D), lambda i: (0, 0)),
        out_shape=jax.ShapeDtypeStruct((G, D), jnp.float32),
        scratch_shapes=[pltpu.VMEM((G, D), jnp.float32)],
    )(h, batchf, lin_W, linb2)


def _lr(v, slope):
    return jnp.where(v >= 0, v, slope * v)


def kernel(x, edge_index, edge_attr, batch, Wn, bn, We, be,
           c1_W, c1_We, c1_as, c1_ad, c1_ae, c1_b,
           c2_W, c2_We, c2_as, c2_ad, c2_ae, c2_b,
           c3_W, c3_We, c3_as, c3_ad, c3_ae, c3_b,
           lin_W, lin_b):
    src_i, dst = edge_index[0], edge_index[1]
    layers = [(c1_W, c1_as[0, 0], c1_ad[0, 0], c1_b),
              (c2_W, c2_as[0, 0], c2_ad[0, 0], c2_b),
              (c3_W, c3_as[0, 0], c3_ad[0, 0], c3_b)]
    V = jnp.stack([c1_We @ c1_ae[0, 0], c2_We @ c2_ae[0, 0],
                   c3_We @ c3_ae[0, 0]], axis=1)
    Vp = jnp.pad(V, ((0, 0), (0, 5)))
    ae8 = _edge_logits(edge_attr, We, be[None, :], Vp)
    ae_cols = [jnp.asarray(ae8[:, i], jnp.float32) for i in range(3)]

    degp, aesp = _sc_prologue(dst, *ae_cols)
    degpT = degp.reshape(NW, N).T
    alpT = [aesp.reshape(3, NW, N)[i].T for i in range(3)]

    h = _proj0(x, Wn, bn[None, :])
    for l, (W, asw, adw, b) in enumerate(layers):
        A8 = jnp.pad(jnp.stack([asw, adw], axis=1), ((0, 0), (0, 6)))
        xh, s8 = _projL(h, W, A8)
        asv = jnp.asarray(s8[:, 0], jnp.float32)
        adv = jnp.asarray(s8[:, 1], jnp.float32)
        denp, exq = _sc_scalar(src_i, dst, ae_cols[l], asv, adv)
        accp, = _sc_rows(src_i, dst, exq, xh)
        denpT = denp.reshape(NW, N).T
        h = _epilogue(accp, denpT, degpT, alpT[l], xh, s8, b[None, :])

    batchf = jnp.broadcast_to(batch.astype(jnp.float32)[:, None], (N, 8))
    return _pool(h, batchf, lin_W, lin_b[None, :])
